# baseline (speedup 1.0000x reference)
import jax
import jax.numpy as jnp
from jax import lax
from jax.experimental import pallas as pl
from jax.experimental.pallas import tpu as pltpu
from jax.experimental.pallas import tpu_sc as plsc

N = 10000
N_PAD = 10240
E = 320000
E_PAD = 327680
K0 = 8000
K1 = 6400
NC = 2
NS = 16
NB_DEG = E_PAD // (NC * NS) // 128
NB_SC = E_PAD // NS // 128
ROWS_PER_SUB = N_PAD // NS


def _deg_body(src_hbm, dst_hbm, kept_hbm, dega_hbm, degb_hbm,
              src_v, dst_v, kept_v, rows_v, acc_sh):
    c = lax.axis_index("c")
    s = lax.axis_index("s")
    wid = s * NC + c

    def zrow(r, carry):
        rows_v[r, :] = jnp.zeros((16,), jnp.float32)
        return carry
    lax.fori_loop(0, 128, zrow, 0)
    for kk in range(ROWS_PER_SUB // 128):
        pltpu.sync_copy(rows_v, acc_sh.at[pl.ds(s * ROWS_PER_SUB + kk * 128, 128)])
    pltpu.sync_copy(src_hbm.at[wid], src_v)
    pltpu.sync_copy(dst_hbm.at[wid], dst_v)
    pltpu.sync_copy(kept_hbm, kept_v)
    plsc.subcore_barrier()

    iota16 = lax.broadcasted_iota(jnp.int32, (16,), 0)
    zero16 = jnp.zeros((16,), jnp.int32)

    def body(j, carry):
        for t in range(8):
            s16 = src_v[j, pl.ds(16 * t, 16)]
            d16 = dst_v[j, pl.ds(16 * t, 16)]
            sv = plsc.load_gather(kept_v, [s16])
            dv = plsc.load_gather(kept_v, [d16])
            plsc.store_scatter(rows_v, [iota16 + (16 * t), zero16], sv * dv)
        pltpu.sync_copy(rows_v, acc_sh.at[dst_v.at[j]], add=True)
        return carry
    lax.fori_loop(0, NB_DEG, body, 0)
    plsc.subcore_barrier()

    @pl.when(c == 0)
    def _():
        pltpu.sync_copy(acc_sh.at[pl.ds(s * ROWS_PER_SUB, ROWS_PER_SUB)],
                        dega_hbm.at[pl.ds(s * ROWS_PER_SUB, ROWS_PER_SUB)])

    @pl.when(c == 1)
    def _():
        pltpu.sync_copy(acc_sh.at[pl.ds(s * ROWS_PER_SUB, ROWS_PER_SUB)],
                        degb_hbm.at[pl.ds(s * ROWS_PER_SUB, ROWS_PER_SUB)])


def _make_deg():
    mesh = plsc.VectorSubcoreMesh(core_axis_name="c", subcore_axis_name="s")
    return pl.kernel(
        _deg_body,
        mesh=mesh,
        compiler_params=pltpu.CompilerParams(needs_layout_passes=False,
                                             use_tc_tiling_on_sc=False),
        out_type=[jax.ShapeDtypeStruct((N_PAD, 16), jnp.float32),
                  jax.ShapeDtypeStruct((N_PAD, 16), jnp.float32)],
        scratch_types=[
            pltpu.VMEM((NB_DEG, 128), jnp.int32),
            pltpu.VMEM((NB_DEG, 128), jnp.int32),
            pltpu.VMEM((N_PAD,), jnp.float32),
            pltpu.VMEM((128, 16), jnp.float32),
            pltpu.VMEM_SHARED((N_PAD, 16), jnp.float32),
        ],
    )


def _zero_acc_slice(buf_v, acc_sh, s, dh):
    def zrow(r, carry):
        for t in range(dh // 16):
            buf_v[r, pl.ds(16 * t, 16)] = jnp.zeros((16,), jnp.float32)
        return carry
    lax.fori_loop(0, 128, zrow, 0)
    for kk in range(ROWS_PER_SUB // 128):
        pltpu.sync_copy(buf_v, acc_sh.at[pl.ds(s * ROWS_PER_SUB + kk * 128, 128)])


def _agg_loop(nb, g_ref, src_v, dst_v, buf_a, buf_b, sem_a, sem_b,
              ssem_a, ssem_b, acc_sh):
    pltpu.make_async_copy(g_ref.at[src_v.at[0]], buf_a, sem_a).start()
    pltpu.make_async_copy(g_ref.at[src_v.at[1]], buf_b, sem_b).start()

    def body(i, carry):
        j = 2 * i
        pltpu.make_async_copy(g_ref.at[src_v.at[j]], buf_a, sem_a).wait()
        pltpu.make_async_copy(buf_a, acc_sh.at[dst_v.at[j]],
                              ssem_a).start(add=True)
        pltpu.make_async_copy(g_ref.at[src_v.at[j + 1]], buf_b, sem_b).wait()
        pltpu.make_async_copy(buf_b, acc_sh.at[dst_v.at[j + 1]],
                              ssem_b).start(add=True)
        pltpu.make_async_copy(buf_a, acc_sh.at[dst_v.at[j]], ssem_a).wait()

        @pl.when(j + 2 < nb)
        def _():
            pltpu.make_async_copy(g_ref.at[src_v.at[j + 2]], buf_a, sem_a).start()
        pltpu.make_async_copy(buf_b, acc_sh.at[dst_v.at[j + 1]], ssem_b).wait()

        @pl.when(j + 3 < nb)
        def _():
            pltpu.make_async_copy(g_ref.at[src_v.at[j + 3]], buf_b, sem_b).start()
        return carry
    lax.fori_loop(0, nb // 2, body, 0)


def _scatter_body(dh, ga_hbm, gb_hbm, src_hbm, dst_hbm, agga_hbm, aggb_hbm,
                  src_v, dst_v, buf_a, buf_b, acc_sh, sem_a, sem_b,
                  ssem_a, ssem_b):
    c = lax.axis_index("c")
    s = lax.axis_index("s")
    _zero_acc_slice(buf_a, acc_sh, s, dh)
    pltpu.sync_copy(src_hbm.at[s], src_v)
    pltpu.sync_copy(dst_hbm.at[s], dst_v)
    plsc.subcore_barrier()

    @pl.when(c == 0)
    def _():
        _agg_loop(NB_SC, ga_hbm, src_v, dst_v, buf_a, buf_b, sem_a, sem_b,
                  ssem_a, ssem_b, acc_sh)

    @pl.when(c == 1)
    def _():
        _agg_loop(NB_SC, gb_hbm, src_v, dst_v, buf_a, buf_b, sem_a, sem_b,
                  ssem_a, ssem_b, acc_sh)
    plsc.subcore_barrier()

    @pl.when(c == 0)
    def _():
        pltpu.sync_copy(acc_sh.at[pl.ds(s * ROWS_PER_SUB, ROWS_PER_SUB)],
                        agga_hbm.at[pl.ds(s * ROWS_PER_SUB, ROWS_PER_SUB)])

    @pl.when(c == 1)
    def _():
        pltpu.sync_copy(acc_sh.at[pl.ds(s * ROWS_PER_SUB, ROWS_PER_SUB)],
                        aggb_hbm.at[pl.ds(s * ROWS_PER_SUB, ROWS_PER_SUB)])


def _make_scatter(dh):
    mesh = plsc.VectorSubcoreMesh(core_axis_name="c", subcore_axis_name="s")
    return pl.kernel(
        lambda *refs: _scatter_body(dh, *refs),
        mesh=mesh,
        compiler_params=pltpu.CompilerParams(needs_layout_passes=False,
                                             use_tc_tiling_on_sc=False),
        out_type=[jax.ShapeDtypeStruct((N_PAD, dh), jnp.float32),
                  jax.ShapeDtypeStruct((N_PAD, dh), jnp.float32)],
        scratch_types=[
            pltpu.VMEM((NB_SC, 128), jnp.int32),
            pltpu.VMEM((NB_SC, 128), jnp.int32),
            pltpu.VMEM((128, dh), jnp.float32),
            pltpu.VMEM((128, dh), jnp.float32),
            pltpu.VMEM_SHARED((N_PAD, dh), jnp.float32),
            pltpu.SemaphoreType.DMA,
            pltpu.SemaphoreType.DMA,
            pltpu.SemaphoreType.DMA,
            pltpu.SemaphoreType.DMA,
        ],
    )


def _scatter2_body(ga0, ga1, gb0, gb1, src_hbm, dst_hbm,
                   oa0, oa1, ob0, ob1, src_v, dst_v, buf_a, buf_b, acc_sh,
                   sem_a, sem_b, ssem_a, ssem_b):
    c = lax.axis_index("c")
    s = lax.axis_index("s")
    pltpu.sync_copy(src_hbm.at[s], src_v)
    pltpu.sync_copy(dst_hbm.at[s], dst_v)

    def phase(g_ref, o_ref):
        _zero_acc_slice(buf_a, acc_sh, s, 64)
        plsc.subcore_barrier()
        _agg_loop(NB_SC, g_ref, src_v, dst_v, buf_a, buf_b, sem_a, sem_b,
                  ssem_a, ssem_b, acc_sh)
        plsc.subcore_barrier()
        pltpu.sync_copy(acc_sh.at[pl.ds(s * ROWS_PER_SUB, ROWS_PER_SUB)],
                        o_ref.at[pl.ds(s * ROWS_PER_SUB, ROWS_PER_SUB)])
        plsc.subcore_barrier()

    @pl.when(c == 0)
    def _():
        phase(ga0, oa0)
        phase(ga1, oa1)

    @pl.when(c == 1)
    def _():
        phase(gb0, ob0)
        phase(gb1, ob1)


def _make_scatter2():
    mesh = plsc.VectorSubcoreMesh(core_axis_name="c", subcore_axis_name="s")
    return pl.kernel(
        _scatter2_body,
        mesh=mesh,
        compiler_params=pltpu.CompilerParams(needs_layout_passes=False,
                                             use_tc_tiling_on_sc=False),
        out_type=[jax.ShapeDtypeStruct((N_PAD, 64), jnp.float32)] * 4,
        scratch_types=[
            pltpu.VMEM((NB_SC, 128), jnp.int32),
            pltpu.VMEM((NB_SC, 128), jnp.int32),
            pltpu.VMEM((128, 64), jnp.float32),
            pltpu.VMEM((128, 64), jnp.float32),
            pltpu.VMEM_SHARED((N_PAD, 64), jnp.float32),
            pltpu.SemaphoreType.DMA,
            pltpu.SemaphoreType.DMA,
            pltpu.SemaphoreType.DMA,
            pltpu.SemaphoreType.DMA,
        ],
    )


def _kth_bits(bits, k):
    def step(_, lohi):
        lo, hi = lohi
        mid = lo + (hi - lo) // 2
        cnt = jnp.sum((bits >= mid).astype(jnp.int32))
        ok = cnt >= k
        return jnp.where(ok, mid, lo), jnp.where(ok, hi, mid)
    lo, _ = lax.fori_loop(0, 31, step, (jnp.int32(0), jnp.int32(0x7F800001)))
    return lo


def _tc_a_body(x_ref, w_ref, da_ref, db_ref, ga_ref, gb_ref, dk_ref):
    x = x_ref[...]
    h = lax.dot_general(x, w_ref[...], (((2,), (0,)), ((), ())),
                        preferred_element_type=jnp.float32)
    deg = 1.0 + da_ref[...] + db_ref[...]
    dk = lax.rsqrt(deg)
    dk_ref[...] = dk
    g = h * dk[:, :, None]
    ga_ref[...] = g[:, :, :64]
    gb_ref[...] = g[:, :, 64:]


def _tc_d1_body(ga_ref, gb_ref, aa_ref, ab_ref, dk_ref, b_ref, p_ref,
                x1m_ref, kept_ref):
    g = jnp.concatenate([ga_ref[...], gb_ref[...]], axis=2)
    agg = jnp.concatenate([aa_ref[...], ab_ref[...]], axis=2)
    dk = dk_ref[...]
    x1 = jax.nn.relu(dk[:, :, None] * (agg + g) + b_ref[...])
    p = p_ref[...]
    pn = p * lax.rsqrt(jnp.sum(p * p))
    sc = lax.dot_general(x1, pn, (((2,), (1,)), ((), ())),
                         preferred_element_type=jnp.float32)[..., 0]
    score = jax.nn.relu(sc)
    ridx = lax.broadcasted_iota(jnp.int32, (80, 128), 0)
    cidx = lax.broadcasted_iota(jnp.int32, (80, 128), 1)
    valid = (ridx * 128 + cidx) < N
    score = jnp.where(valid, score, 0.0)
    bits = lax.bitcast_convert_type(score, jnp.int32)
    lo = _kth_bits(bits, K0)
    pos = bits > 0
    need = (K0 - jnp.sum(pos.astype(jnp.int32))).astype(jnp.float32)
    z = jnp.where((bits == 0) & valid, 1.0, 0.0)
    rs = jnp.sum(z, axis=1, keepdims=True)
    ri = lax.broadcasted_iota(jnp.int32, (80, 80), 0)
    ci = lax.broadcasted_iota(jnp.int32, (80, 80), 1)
    tri_r = jnp.where(ri > ci, 1.0, 0.0)
    pre_rows = jnp.dot(tri_r, rs, preferred_element_type=jnp.float32)
    ai = lax.broadcasted_iota(jnp.int32, (128, 128), 0)
    bi = lax.broadcasted_iota(jnp.int32, (128, 128), 1)
    tri_c = jnp.where(ai < bi, 1.0, 0.0)
    within = jnp.dot(z, tri_c, preferred_element_type=jnp.float32)
    zrank = pre_rows + within
    kept_b = (bits >= jnp.maximum(lo, 1)) | ((z > 0.0) & (zrank < need))
    kept = jnp.where(kept_b, 1.0, 0.0)
    kept_ref[...] = kept
    x1m_ref[...] = x1 * (score * kept)[:, :, None]


def _tc_f_body(x1m_ref, w_ref, da_ref, db_ref, kept_ref,
               gh0_ref, gh1_ref, dk_ref):
    deg = 1.0 + da_ref[...] + db_ref[...]
    dk = lax.rsqrt(deg) * kept_ref[...]
    dk_ref[...] = dk
    x1m = x1m_ref[...]
    w = w_ref[...]
    h0 = lax.dot_general(x1m, w[:, :128], (((2,), (0,)), ((), ())),
                         preferred_element_type=jnp.float32)
    gh0_ref[...] = h0 * dk[:, :, None]
    h1 = lax.dot_general(x1m, w[:, 128:], (((2,), (0,)), ((), ())),
                         preferred_element_type=jnp.float32)
    gh1_ref[...] = h1 * dk[:, :, None]


def _tc_h_body(gh0_ref, gh1_ref, ah0_ref, ah1_ref,
               dk_ref, kept_ref, b_ref, p_ref,
               l1w_ref, l1b_ref, l2wa_ref, l2wb_ref, l2b_ref, sc_ref,
               pred_ref, xl1_ref):
    dk = dk_ref[...]
    p = p_ref[...]
    pn = p * lax.rsqrt(jnp.sum(p * p))
    g_refs = (gh0_ref, gh1_ref)
    a_refs = (ah0_ref, ah1_ref)

    def xhalf(q):
        return jax.nn.relu(dk[:, :, None] * (a_refs[q][...] + g_refs[q][...])
                           + b_ref[0:1, 128 * q:128 * (q + 1)])

    s2 = jnp.zeros((80, 128), jnp.float32)
    for q in range(2):
        pq = pn[0:1, 128 * q:128 * (q + 1)]
        s2 = s2 + lax.dot_general(xhalf(q), pq, (((2,), (1,)), ((), ())),
                                  preferred_element_type=jnp.float32)[..., 0]
    score = jax.nn.relu(s2) * kept_ref[...]
    bits = lax.bitcast_convert_type(score, jnp.int32)
    lo = _kth_bits(bits, K1)
    sel = bits >= jnp.maximum(lo, 1)
    wgt = jnp.where(sel, score, 0.0)
    pooled_q = []
    for q in range(2):
        part = jnp.sum(xhalf(q) * wgt[:, :, None], axis=0)
        pooled_q.append(jnp.sum(part, axis=0, keepdims=True) / K1)
    pooled = jnp.concatenate(pooled_q, axis=1)
    xl1 = jax.nn.relu(jnp.dot(pooled, l1w_ref[...],
                              preferred_element_type=jnp.float32) + l1b_ref[...])
    sexv = sc_ref[0, 0]
    cagv = sc_ref[0, 1]
    logits = (jnp.dot(xl1, l2wa_ref[...], preferred_element_type=jnp.float32)
              + sexv * l2wb_ref[0:1, :] + cagv * l2wb_ref[1:2, :] + l2b_ref[...])
    lane = lax.broadcasted_iota(jnp.int32, (1, 128), 1)
    neg = jnp.float32(-3.0e38)
    ml = jnp.where(lane < 5, logits, neg)
    m = jnp.max(ml)
    e = jnp.where(lane < 5, jnp.exp(ml - m), 0.0)
    prob = e / jnp.sum(e)
    pred_ref[...] = jnp.broadcast_to(prob, (8, 128))
    xl1_ref[...] = jnp.broadcast_to(xl1, (8, 256))


def _sds(shape):
    return jax.ShapeDtypeStruct(shape, jnp.float32)


_tc_a = pl.pallas_call(_tc_a_body,
                       out_shape=[_sds((80, 128, 64)), _sds((80, 128, 64)),
                                  _sds((80, 128))])
_tc_d1 = pl.pallas_call(_tc_d1_body,
                        out_shape=[_sds((80, 128, 128)), _sds((80, 128))])
_tc_f = pl.pallas_call(_tc_f_body,
                       out_shape=[_sds((80, 128, 128)), _sds((80, 128, 128)),
                                  _sds((80, 128))])
_tc_h = pl.pallas_call(_tc_h_body,
                       out_shape=[_sds((8, 128)), _sds((8, 256))])

_deg_call = _make_deg()
_scatter64 = _make_scatter(64)
_scatter2 = _make_scatter2()


def kernel(x, edge_index, edge_attr, batch, sex, cag, conv_W0, conv_b0,
           pool_p0, conv_W1, conv_b1, pool_p1, lin1_W, lin1_b, lin2_W, lin2_b):
    f32 = jnp.float32
    x_p = jnp.pad(x, ((0, N_PAD - N), (0, 0))).reshape(80, 128, 128)
    padi = jnp.full((E_PAD - E,), N_PAD - 1, jnp.int32)
    src_p = jnp.concatenate([edge_index[0], padi])
    dst_p = jnp.concatenate([edge_index[1], padi])
    src_deg = src_p.reshape(NC * NS, NB_DEG, 128)
    dst_deg = dst_p.reshape(NC * NS, NB_DEG, 128)
    src_sc = src_p.reshape(NS, NB_SC, 128)
    dst_sc = dst_p.reshape(NS, NB_SC, 128)
    ones_k = (jnp.arange(N_PAD) < N).astype(f32)

    dega, degb = _deg_call(src_deg, dst_deg, ones_k)
    d0a = dega[:, 0].reshape(80, 128)
    d0b = degb[:, 0].reshape(80, 128)
    g0a, g0b, dk0 = _tc_a(x_p, conv_W0, d0a, d0b)
    agg0a, agg0b = _scatter64(g0a.reshape(N_PAD, 64), g0b.reshape(N_PAD, 64),
                              src_sc, dst_sc)
    x1m, kept0 = _tc_d1(g0a, g0b, agg0a.reshape(80, 128, 64),
                        agg0b.reshape(80, 128, 64), dk0,
                        conv_b0.reshape(1, 128), pool_p0.reshape(1, 128))
    deg1a, deg1b = _deg_call(src_deg, dst_deg, kept0.reshape(N_PAD))
    d1a = deg1a[:, 0].reshape(80, 128)
    d1b = deg1b[:, 0].reshape(80, 128)
    gh0, gh1, dk1 = _tc_f(x1m, conv_W1, d1a, d1b, kept0)
    gh0f = gh0.reshape(N_PAD, 128)
    gh1f = gh1.reshape(N_PAD, 128)
    a10, a11, a12, a13 = _scatter2(gh0f[:, :64], gh0f[:, 64:],
                                   gh1f[:, :64], gh1f[:, 64:],
                                   src_sc, dst_sc)
    ah0 = jnp.concatenate([a10, a11], axis=1).reshape(80, 128, 128)
    ah1 = jnp.concatenate([a12, a13], axis=1).reshape(80, 128, 128)
    l2wa = jnp.pad(lin2_W[:256], ((0, 0), (0, 123)))
    l2wb = jnp.pad(lin2_W[256:], ((0, 0), (0, 123)))
    l2bp = jnp.pad(lin2_b.reshape(1, 5), ((0, 0), (0, 123)))
    scv = jnp.pad(jnp.concatenate([sex, cag]).reshape(1, 2), ((0, 0), (0, 126)))
    pred8, xl18 = _tc_h(gh0, gh1, ah0, ah1, dk1, kept0,
                        conv_b1.reshape(1, 256), pool_p1.reshape(1, 256),
                        lin1_W, lin1_b.reshape(1, 256), l2wa, l2wb, l2bp, scv)
    return pred8[0:1, 0:5], xl18[0:1, :]


# speedup vs baseline: 1.0698x; 1.0698x over previous
import jax
import jax.numpy as jnp
from jax import lax
from jax.experimental import pallas as pl
from jax.experimental.pallas import tpu as pltpu
from jax.experimental.pallas import tpu_sc as plsc

N = 10000
N_PAD = 10240
E = 320000
E_PAD = 327680
K0 = 8000
K1 = 6400
NC = 2
NS = 16
NB_DEG = E_PAD // (NC * NS) // 128
NB_SC = E_PAD // NS // 128
ROWS_PER_SUB = N_PAD // NS


def _deg_body(src_hbm, dst_hbm, kept_hbm, dega_hbm, degb_hbm,
              src_v, dst_v, kept_v, rows_v, acc_sh):
    c = lax.axis_index("c")
    s = lax.axis_index("s")
    wid = s * NC + c

    def zrow(r, carry):
        rows_v[r, :] = jnp.zeros((16,), jnp.float32)
        return carry
    lax.fori_loop(0, 128, zrow, 0)
    for kk in range(ROWS_PER_SUB // 128):
        pltpu.sync_copy(rows_v, acc_sh.at[pl.ds(s * ROWS_PER_SUB + kk * 128, 128)])
    pltpu.sync_copy(src_hbm.at[wid], src_v)
    pltpu.sync_copy(dst_hbm.at[wid], dst_v)
    pltpu.sync_copy(kept_hbm, kept_v)
    plsc.subcore_barrier()

    iota16 = lax.broadcasted_iota(jnp.int32, (16,), 0)
    zero16 = jnp.zeros((16,), jnp.int32)

    def body(j, carry):
        for t in range(8):
            s16 = src_v[j, pl.ds(16 * t, 16)]
            d16 = dst_v[j, pl.ds(16 * t, 16)]
            sv = plsc.load_gather(kept_v, [s16])
            dv = plsc.load_gather(kept_v, [d16])
            plsc.store_scatter(rows_v, [iota16 + (16 * t), zero16], sv * dv)
        pltpu.sync_copy(rows_v, acc_sh.at[dst_v.at[j]], add=True)
        return carry
    lax.fori_loop(0, NB_DEG, body, 0)
    plsc.subcore_barrier()

    @pl.when(c == 0)
    def _():
        pltpu.sync_copy(acc_sh.at[pl.ds(s * ROWS_PER_SUB, ROWS_PER_SUB)],
                        dega_hbm.at[pl.ds(s * ROWS_PER_SUB, ROWS_PER_SUB)])

    @pl.when(c == 1)
    def _():
        pltpu.sync_copy(acc_sh.at[pl.ds(s * ROWS_PER_SUB, ROWS_PER_SUB)],
                        degb_hbm.at[pl.ds(s * ROWS_PER_SUB, ROWS_PER_SUB)])


def _make_deg():
    mesh = plsc.VectorSubcoreMesh(core_axis_name="c", subcore_axis_name="s")
    return pl.kernel(
        _deg_body,
        mesh=mesh,
        compiler_params=pltpu.CompilerParams(needs_layout_passes=False,
                                             use_tc_tiling_on_sc=False),
        out_type=[jax.ShapeDtypeStruct((N_PAD, 16), jnp.float32),
                  jax.ShapeDtypeStruct((N_PAD, 16), jnp.float32)],
        scratch_types=[
            pltpu.VMEM((NB_DEG, 128), jnp.int32),
            pltpu.VMEM((NB_DEG, 128), jnp.int32),
            pltpu.VMEM((N_PAD,), jnp.float32),
            pltpu.VMEM((128, 16), jnp.float32),
            pltpu.VMEM_SHARED((N_PAD, 16), jnp.float32),
        ],
    )


def _zero_acc_slice(buf_v, acc_sh, s, dh):
    def zrow(r, carry):
        for t in range(dh // 16):
            buf_v[r, pl.ds(16 * t, 16)] = jnp.zeros((16,), jnp.float32)
        return carry
    lax.fori_loop(0, 128, zrow, 0)
    for kk in range(ROWS_PER_SUB // 128):
        pltpu.sync_copy(buf_v, acc_sh.at[pl.ds(s * ROWS_PER_SUB + kk * 128, 128)])


def _agg_loop(nb, g_ref, src_v, dst_v, buf_a, buf_b, sem_a, sem_b, acc_sh):
    pltpu.make_async_copy(g_ref.at[src_v.at[0]], buf_a, sem_a).start()
    pltpu.make_async_copy(g_ref.at[src_v.at[1]], buf_b, sem_b).start()

    def body(i, carry):
        j = 2 * i
        pltpu.make_async_copy(g_ref.at[src_v.at[j]], buf_a, sem_a).wait()
        pltpu.sync_copy(buf_a, acc_sh.at[dst_v.at[j]], add=True)

        @pl.when(j + 2 < nb)
        def _():
            pltpu.make_async_copy(g_ref.at[src_v.at[j + 2]], buf_a, sem_a).start()
        pltpu.make_async_copy(g_ref.at[src_v.at[j + 1]], buf_b, sem_b).wait()
        pltpu.sync_copy(buf_b, acc_sh.at[dst_v.at[j + 1]], add=True)

        @pl.when(j + 3 < nb)
        def _():
            pltpu.make_async_copy(g_ref.at[src_v.at[j + 3]], buf_b, sem_b).start()
        return carry
    lax.fori_loop(0, nb // 2, body, 0)


def _scatter_body(dh, ga_hbm, gb_hbm, src_hbm, dst_hbm, agga_hbm, aggb_hbm,
                  src_v, dst_v, buf_a, buf_b, acc_sh, sem_a, sem_b):
    c = lax.axis_index("c")
    s = lax.axis_index("s")
    _zero_acc_slice(buf_a, acc_sh, s, dh)
    pltpu.sync_copy(src_hbm.at[s], src_v)
    pltpu.sync_copy(dst_hbm.at[s], dst_v)
    plsc.subcore_barrier()

    @pl.when(c == 0)
    def _():
        _agg_loop(NB_SC, ga_hbm, src_v, dst_v, buf_a, buf_b, sem_a, sem_b,
                  acc_sh)

    @pl.when(c == 1)
    def _():
        _agg_loop(NB_SC, gb_hbm, src_v, dst_v, buf_a, buf_b, sem_a, sem_b,
                  acc_sh)
    plsc.subcore_barrier()

    @pl.when(c == 0)
    def _():
        pltpu.sync_copy(acc_sh.at[pl.ds(s * ROWS_PER_SUB, ROWS_PER_SUB)],
                        agga_hbm.at[pl.ds(s * ROWS_PER_SUB, ROWS_PER_SUB)])

    @pl.when(c == 1)
    def _():
        pltpu.sync_copy(acc_sh.at[pl.ds(s * ROWS_PER_SUB, ROWS_PER_SUB)],
                        aggb_hbm.at[pl.ds(s * ROWS_PER_SUB, ROWS_PER_SUB)])


def _make_scatter(dh):
    mesh = plsc.VectorSubcoreMesh(core_axis_name="c", subcore_axis_name="s")
    return pl.kernel(
        lambda *refs: _scatter_body(dh, *refs),
        mesh=mesh,
        compiler_params=pltpu.CompilerParams(needs_layout_passes=False,
                                             use_tc_tiling_on_sc=False),
        out_type=[jax.ShapeDtypeStruct((N_PAD, dh), jnp.float32),
                  jax.ShapeDtypeStruct((N_PAD, dh), jnp.float32)],
        scratch_types=[
            pltpu.VMEM((NB_SC, 128), jnp.int32),
            pltpu.VMEM((NB_SC, 128), jnp.int32),
            pltpu.VMEM((128, dh), jnp.float32),
            pltpu.VMEM((128, dh), jnp.float32),
            pltpu.VMEM_SHARED((N_PAD, dh), jnp.float32),
            pltpu.SemaphoreType.DMA,
            pltpu.SemaphoreType.DMA,
        ],
    )


def _scatter2_body(ga0, ga1, gb0, gb1, src_hbm, dst_hbm,
                   oa0, oa1, ob0, ob1, src_v, dst_v, buf_a, buf_b, acc_sh,
                   sem_a, sem_b):
    c = lax.axis_index("c")
    s = lax.axis_index("s")
    pltpu.sync_copy(src_hbm.at[s], src_v)
    pltpu.sync_copy(dst_hbm.at[s], dst_v)

    def phase(g_ref, o_ref):
        _zero_acc_slice(buf_a, acc_sh, s, 64)
        plsc.subcore_barrier()
        _agg_loop(NB_SC, g_ref, src_v, dst_v, buf_a, buf_b, sem_a, sem_b,
                  acc_sh)
        plsc.subcore_barrier()
        pltpu.sync_copy(acc_sh.at[pl.ds(s * ROWS_PER_SUB, ROWS_PER_SUB)],
                        o_ref.at[pl.ds(s * ROWS_PER_SUB, ROWS_PER_SUB)])
        plsc.subcore_barrier()

    @pl.when(c == 0)
    def _():
        phase(ga0, oa0)
        phase(ga1, oa1)

    @pl.when(c == 1)
    def _():
        phase(gb0, ob0)
        phase(gb1, ob1)


def _make_scatter2():
    mesh = plsc.VectorSubcoreMesh(core_axis_name="c", subcore_axis_name="s")
    return pl.kernel(
        _scatter2_body,
        mesh=mesh,
        compiler_params=pltpu.CompilerParams(needs_layout_passes=False,
                                             use_tc_tiling_on_sc=False),
        out_type=[jax.ShapeDtypeStruct((N_PAD, 64), jnp.float32)] * 4,
        scratch_types=[
            pltpu.VMEM((NB_SC, 128), jnp.int32),
            pltpu.VMEM((NB_SC, 128), jnp.int32),
            pltpu.VMEM((128, 64), jnp.float32),
            pltpu.VMEM((128, 64), jnp.float32),
            pltpu.VMEM_SHARED((N_PAD, 64), jnp.float32),
            pltpu.SemaphoreType.DMA,
            pltpu.SemaphoreType.DMA,
        ],
    )


def _kth_bits(bits, k):
    def step(_, lohi):
        lo, hi = lohi
        mid = lo + (hi - lo) // 2
        cnt = jnp.sum((bits >= mid).astype(jnp.int32))
        ok = cnt >= k
        return jnp.where(ok, mid, lo), jnp.where(ok, hi, mid)
    lo, _ = lax.fori_loop(0, 31, step, (jnp.int32(0), jnp.int32(0x7F800001)))
    return lo


def _tc_a_body(x_ref, w_ref, da_ref, db_ref, ga_ref, gb_ref, dk_ref):
    x = x_ref[...]
    h = lax.dot_general(x, w_ref[...], (((2,), (0,)), ((), ())),
                        preferred_element_type=jnp.float32)
    deg = 1.0 + da_ref[...] + db_ref[...]
    dk = lax.rsqrt(deg)
    dk_ref[...] = dk
    g = h * dk[:, :, None]
    ga_ref[...] = g[:, :, :64]
    gb_ref[...] = g[:, :, 64:]


def _tc_d1_body(ga_ref, gb_ref, aa_ref, ab_ref, dk_ref, b_ref, p_ref,
                x1m_ref, kept_ref):
    g = jnp.concatenate([ga_ref[...], gb_ref[...]], axis=2)
    agg = jnp.concatenate([aa_ref[...], ab_ref[...]], axis=2)
    dk = dk_ref[...]
    x1 = jax.nn.relu(dk[:, :, None] * (agg + g) + b_ref[...])
    p = p_ref[...]
    pn = p * lax.rsqrt(jnp.sum(p * p))
    sc = lax.dot_general(x1, pn, (((2,), (1,)), ((), ())),
                         preferred_element_type=jnp.float32)[..., 0]
    score = jax.nn.relu(sc)
    ridx = lax.broadcasted_iota(jnp.int32, (80, 128), 0)
    cidx = lax.broadcasted_iota(jnp.int32, (80, 128), 1)
    valid = (ridx * 128 + cidx) < N
    score = jnp.where(valid, score, 0.0)
    bits = lax.bitcast_convert_type(score, jnp.int32)
    lo = _kth_bits(bits, K0)
    pos = bits > 0
    need = (K0 - jnp.sum(pos.astype(jnp.int32))).astype(jnp.float32)
    z = jnp.where((bits == 0) & valid, 1.0, 0.0)
    rs = jnp.sum(z, axis=1, keepdims=True)
    ri = lax.broadcasted_iota(jnp.int32, (80, 80), 0)
    ci = lax.broadcasted_iota(jnp.int32, (80, 80), 1)
    tri_r = jnp.where(ri > ci, 1.0, 0.0)
    pre_rows = jnp.dot(tri_r, rs, preferred_element_type=jnp.float32)
    ai = lax.broadcasted_iota(jnp.int32, (128, 128), 0)
    bi = lax.broadcasted_iota(jnp.int32, (128, 128), 1)
    tri_c = jnp.where(ai < bi, 1.0, 0.0)
    within = jnp.dot(z, tri_c, preferred_element_type=jnp.float32)
    zrank = pre_rows + within
    kept_b = (bits >= jnp.maximum(lo, 1)) | ((z > 0.0) & (zrank < need))
    kept = jnp.where(kept_b, 1.0, 0.0)
    kept_ref[...] = kept
    x1m_ref[...] = x1 * (score * kept)[:, :, None]


def _tc_f_body(x1m_ref, w_ref, da_ref, db_ref, kept_ref,
               gh0_ref, gh1_ref, dk_ref):
    deg = 1.0 + da_ref[...] + db_ref[...]
    dk = lax.rsqrt(deg) * kept_ref[...]
    dk_ref[...] = dk
    x1m = x1m_ref[...]
    w = w_ref[...]
    h0 = lax.dot_general(x1m, w[:, :128], (((2,), (0,)), ((), ())),
                         preferred_element_type=jnp.float32)
    gh0_ref[...] = h0 * dk[:, :, None]
    h1 = lax.dot_general(x1m, w[:, 128:], (((2,), (0,)), ((), ())),
                         preferred_element_type=jnp.float32)
    gh1_ref[...] = h1 * dk[:, :, None]


def _tc_h_body(gh0_ref, gh1_ref, ah0_ref, ah1_ref,
               dk_ref, kept_ref, b_ref, p_ref,
               l1w_ref, l1b_ref, l2wa_ref, l2wb_ref, l2b_ref, sc_ref,
               pred_ref, xl1_ref):
    dk = dk_ref[...]
    p = p_ref[...]
    pn = p * lax.rsqrt(jnp.sum(p * p))
    g_refs = (gh0_ref, gh1_ref)
    a_refs = (ah0_ref, ah1_ref)

    def xhalf(q):
        return jax.nn.relu(dk[:, :, None] * (a_refs[q][...] + g_refs[q][...])
                           + b_ref[0:1, 128 * q:128 * (q + 1)])

    s2 = jnp.zeros((80, 128), jnp.float32)
    for q in range(2):
        pq = pn[0:1, 128 * q:128 * (q + 1)]
        s2 = s2 + lax.dot_general(xhalf(q), pq, (((2,), (1,)), ((), ())),
                                  preferred_element_type=jnp.float32)[..., 0]
    score = jax.nn.relu(s2) * kept_ref[...]
    bits = lax.bitcast_convert_type(score, jnp.int32)
    lo = _kth_bits(bits, K1)
    sel = bits >= jnp.maximum(lo, 1)
    wgt = jnp.where(sel, score, 0.0)
    pooled_q = []
    for q in range(2):
        part = jnp.sum(xhalf(q) * wgt[:, :, None], axis=0)
        pooled_q.append(jnp.sum(part, axis=0, keepdims=True) / K1)
    pooled = jnp.concatenate(pooled_q, axis=1)
    xl1 = jax.nn.relu(jnp.dot(pooled, l1w_ref[...],
                              preferred_element_type=jnp.float32) + l1b_ref[...])
    sexv = sc_ref[0, 0]
    cagv = sc_ref[0, 1]
    logits = (jnp.dot(xl1, l2wa_ref[...], preferred_element_type=jnp.float32)
              + sexv * l2wb_ref[0:1, :] + cagv * l2wb_ref[1:2, :] + l2b_ref[...])
    lane = lax.broadcasted_iota(jnp.int32, (1, 128), 1)
    neg = jnp.float32(-3.0e38)
    ml = jnp.where(lane < 5, logits, neg)
    m = jnp.max(ml)
    e = jnp.where(lane < 5, jnp.exp(ml - m), 0.0)
    prob = e / jnp.sum(e)
    pred_ref[...] = jnp.broadcast_to(prob, (8, 128))
    xl1_ref[...] = jnp.broadcast_to(xl1, (8, 256))


def _sds(shape):
    return jax.ShapeDtypeStruct(shape, jnp.float32)


_tc_a = pl.pallas_call(_tc_a_body,
                       out_shape=[_sds((80, 128, 64)), _sds((80, 128, 64)),
                                  _sds((80, 128))])
_tc_d1 = pl.pallas_call(_tc_d1_body,
                        out_shape=[_sds((80, 128, 128)), _sds((80, 128))])
_tc_f = pl.pallas_call(_tc_f_body,
                       out_shape=[_sds((80, 128, 128)), _sds((80, 128, 128)),
                                  _sds((80, 128))])
_tc_h = pl.pallas_call(_tc_h_body,
                       out_shape=[_sds((8, 128)), _sds((8, 256))])

_deg_call = _make_deg()
_scatter64 = _make_scatter(64)
_scatter2 = _make_scatter2()


def kernel(x, edge_index, edge_attr, batch, sex, cag, conv_W0, conv_b0,
           pool_p0, conv_W1, conv_b1, pool_p1, lin1_W, lin1_b, lin2_W, lin2_b):
    f32 = jnp.float32
    x_p = jnp.pad(x, ((0, N_PAD - N), (0, 0))).reshape(80, 128, 128)
    padi = jnp.full((E_PAD - E,), N_PAD - 1, jnp.int32)
    src_p = jnp.concatenate([edge_index[0], padi])
    dst_p = jnp.concatenate([edge_index[1], padi])
    src_deg = src_p.reshape(NC * NS, NB_DEG, 128)
    dst_deg = dst_p.reshape(NC * NS, NB_DEG, 128)
    src_sc = src_p.reshape(NS, NB_SC, 128)
    dst_sc = dst_p.reshape(NS, NB_SC, 128)
    ones_k = (jnp.arange(N_PAD) < N).astype(f32)

    dega, degb = _deg_call(src_deg, dst_deg, ones_k)
    d0a = dega[:, 0].reshape(80, 128)
    d0b = degb[:, 0].reshape(80, 128)
    g0a, g0b, dk0 = _tc_a(x_p, conv_W0, d0a, d0b)
    agg0a, agg0b = _scatter64(g0a.reshape(N_PAD, 64), g0b.reshape(N_PAD, 64),
                              src_sc, dst_sc)
    x1m, kept0 = _tc_d1(g0a, g0b, agg0a.reshape(80, 128, 64),
                        agg0b.reshape(80, 128, 64), dk0,
                        conv_b0.reshape(1, 128), pool_p0.reshape(1, 128))
    deg1a, deg1b = _deg_call(src_deg, dst_deg, kept0.reshape(N_PAD))
    d1a = deg1a[:, 0].reshape(80, 128)
    d1b = deg1b[:, 0].reshape(80, 128)
    gh0, gh1, dk1 = _tc_f(x1m, conv_W1, d1a, d1b, kept0)
    gh0f = gh0.reshape(N_PAD, 128)
    gh1f = gh1.reshape(N_PAD, 128)
    a10, a11, a12, a13 = _scatter2(gh0f[:, :64], gh0f[:, 64:],
                                   gh1f[:, :64], gh1f[:, 64:],
                                   src_sc, dst_sc)
    ah0 = jnp.concatenate([a10, a11], axis=1).reshape(80, 128, 128)
    ah1 = jnp.concatenate([a12, a13], axis=1).reshape(80, 128, 128)
    l2wa = jnp.pad(lin2_W[:256], ((0, 0), (0, 123)))
    l2wb = jnp.pad(lin2_W[256:], ((0, 0), (0, 123)))
    l2bp = jnp.pad(lin2_b.reshape(1, 5), ((0, 0), (0, 123)))
    scv = jnp.pad(jnp.concatenate([sex, cag]).reshape(1, 2), ((0, 0), (0, 126)))
    pred8, xl18 = _tc_h(gh0, gh1, ah0, ah1, dk1, kept0,
                        conv_b1.reshape(1, 256), pool_p1.reshape(1, 256),
                        lin1_W, lin1_b.reshape(1, 256), l2wa, l2wb, l2bp, scv)
    return pred8[0:1, 0:5], xl18[0:1, :]
